# baseline (speedup 1.0000x reference)
import functools

import jax
import jax.numpy as jnp
from jax import lax
from jax.experimental import pallas as pl
from jax.experimental.pallas import tpu as pltpu
from jax.experimental.pallas import tpu_sc as plsc

SEQ_LEN = 200
EMBED_DIM = 64
BATCH = 4096
VOCAB = 1000000

NC, NS, L = 2, 16, 16
NW = NC * NS
BBLK = BATCH // NW
TROW, TCOL = SEQ_LEN // 8, BATCH // 128
NBUF = 4
SCALE = 8.0


@functools.partial(
    pl.kernel,
    out_type=jax.ShapeDtypeStruct((SEQ_LEN, 8, TCOL, 8, 128), jnp.float32),
    mesh=plsc.VectorSubcoreMesh(core_axis_name="c", subcore_axis_name="s"),
    compiler_params=pltpu.CompilerParams(
        use_tc_tiling_on_sc=False, needs_layout_passes=False),
    scratch_types=[
        [pltpu.VMEM((BBLK,), jnp.int32) for _ in range(NBUF)],
        [pltpu.VMEM((BBLK,), jnp.int32) for _ in range(NBUF)],
        [pltpu.VMEM((BBLK,), jnp.int32) for _ in range(NBUF)],
        [pltpu.VMEM((BBLK, 128), jnp.float32) for _ in range(NBUF)],
        [pltpu.VMEM((8, 8, 128), jnp.float32) for _ in range(NBUF)],
        pltpu.VMEM((SEQ_LEN, EMBED_DIM), jnp.float32),
        [pltpu.SemaphoreType.DMA for _ in range(NBUF)],
        [pltpu.SemaphoreType.DMA for _ in range(NBUF)],
        [pltpu.SemaphoreType.DMA for _ in range(NBUF)],
    ],
)
def _embed_kernel(idx4_hbm, pairs_hbm, pos_hbm, out_hbm,
                  idx_v, idxp_v, hcol_v, rows_v, outb_v, pos_v,
                  isem, gsem, wsem):
    wid = lax.axis_index("s") * NC + lax.axis_index("c")

    pltpu.sync_copy(pos_hbm, pos_v)

    def idx_descr(t, b):
        return pltpu.make_async_copy(
            idx4_hbm.at[t // 8, wid, t % 8], idx_v[b], isem[b])

    def gather_descr(b):
        return pltpu.make_async_copy(
            pairs_hbm.at[idxp_v[b]], rows_v[b], gsem[b])

    def wb_descrs(t, b):
        return [pltpu.make_async_copy(
            outb_v[b].at[er], out_hbm.at[t, er, wid], wsem[b])
            for er in range(8)]

    def prep_and_fire_gather(b):
        for j in range(BBLK // L):
            sl = pl.ds(j * L, L)
            v = idx_v[b][sl]
            idxp_v[b][sl] = lax.shift_right_logical(v, 1)
            hcol_v[b][sl] = lax.shift_left(jnp.bitwise_and(v, 1), 6)
        gather_descr(b).start()

    idx_descr(0, 0).start()
    idx_descr(0, 0).wait()
    prep_and_fire_gather(0)
    idx_descr(1, 1).start()

    iota = lax.iota(jnp.int32, L)

    def outer(o, carry):
        for b in range(NBUF):
            t = o * NBUF + b

            @pl.when(t + 2 < SEQ_LEN)
            def _():
                idx_descr(t + 2, (b + 2) % NBUF).start()

            @pl.when(t + 1 < SEQ_LEN)
            def _():
                nb = (b + 1) % NBUF
                idx_descr(t + 1, nb).wait()
                prep_and_fire_gather(nb)

            gather_descr(b).wait()

            @pl.when(t >= NBUF)
            def _():
                for d in wb_descrs(t - NBUF, b):
                    d.wait()

            t_vec = jnp.full((L,), t, jnp.int32)

            def e_body(e, carry2):
                ps = plsc.load_gather(pos_v, [t_vec, jnp.full((L,), e, jnp.int32)])
                er = lax.shift_right_logical(e, 3)
                es = jnp.bitwise_and(e, 7)
                for j in range(BBLK // L):
                    sl = pl.ds(j * L, L)
                    rowi = iota + (j * L)
                    coli = hcol_v[b][sl] + e
                    vals = plsc.load_gather(rows_v[b], [rowi, coli])
                    outb_v[b][er, es, sl] = vals * SCALE + ps
                return carry2

            lax.fori_loop(0, EMBED_DIM, e_body, 0, unroll=2)

            for d in wb_descrs(t, b):
                d.start()
        return carry

    lax.fori_loop(0, SEQ_LEN // NBUF, outer, 0)

    for b in range(NBUF):
        for d in wb_descrs(SEQ_LEN - NBUF + b, b):
            d.wait()


def kernel(inputs, token_table, pos_table):
    idx4 = inputs.T.reshape(TROW, 8, TCOL, 128).transpose(0, 2, 1, 3)
    pairs = token_table.reshape(VOCAB // 2, 128)
    out5 = _embed_kernel(idx4, pairs, pos_table)
    return out5.transpose(2, 4, 0, 1, 3).reshape(BATCH, SEQ_LEN, EMBED_DIM)


# speedup vs baseline: 1.0023x; 1.0023x over previous
import functools

import jax
import jax.numpy as jnp
from jax import lax
from jax.experimental import pallas as pl
from jax.experimental.pallas import tpu as pltpu
from jax.experimental.pallas import tpu_sc as plsc

SEQ_LEN = 200
EMBED_DIM = 64
BATCH = 4096
VOCAB = 1000000

NC, NS, L = 2, 16, 16
NW = NC * NS
BBLK = BATCH // NW
TROW, TCOL = SEQ_LEN // 8, BATCH // 128
NBUF = 4
SCALE = 8.0


@functools.partial(
    pl.kernel,
    out_type=jax.ShapeDtypeStruct((SEQ_LEN, 8, TCOL, 8, 128), jnp.float32),
    mesh=plsc.VectorSubcoreMesh(core_axis_name="c", subcore_axis_name="s"),
    compiler_params=pltpu.CompilerParams(
        use_tc_tiling_on_sc=False, needs_layout_passes=False),
    scratch_types=[
        [pltpu.VMEM((BBLK,), jnp.int32) for _ in range(NBUF)],
        [pltpu.VMEM((BBLK,), jnp.int32) for _ in range(NBUF)],
        [pltpu.VMEM((BBLK,), jnp.int32) for _ in range(NBUF)],
        [pltpu.VMEM((BBLK, 128), jnp.float32) for _ in range(NBUF)],
        [pltpu.VMEM((8, 8, 128), jnp.float32) for _ in range(NBUF)],
        pltpu.VMEM((SEQ_LEN, EMBED_DIM), jnp.float32),
        [pltpu.SemaphoreType.DMA for _ in range(NBUF)],
        [pltpu.SemaphoreType.DMA for _ in range(NBUF)],
        [pltpu.SemaphoreType.DMA for _ in range(NBUF)],
    ],
)
def _embed_kernel(idx4_hbm, pairs_hbm, pos_hbm, out_hbm,
                  idx_v, idxp_v, hcol_v, rows_v, outb_v, pos_v,
                  isem, gsem, wsem):
    wid = lax.axis_index("s") * NC + lax.axis_index("c")

    pltpu.sync_copy(pos_hbm, pos_v)

    def idx_descr(t, b):
        return pltpu.make_async_copy(
            idx4_hbm.at[t // 8, wid, t % 8], idx_v[b], isem[b])

    def gather_descr(b):
        return pltpu.make_async_copy(
            pairs_hbm.at[idxp_v[b]], rows_v[b], gsem[b])

    def wb_descrs(t, b):
        return [pltpu.make_async_copy(
            outb_v[b].at[er], out_hbm.at[t, er, wid], wsem[b])
            for er in range(8)]

    def prep_and_fire_gather(b):
        for j in range(BBLK // L):
            sl = pl.ds(j * L, L)
            v = idx_v[b][sl]
            idxp_v[b][sl] = lax.shift_right_logical(v, 1)
            hcol_v[b][sl] = lax.shift_left(jnp.bitwise_and(v, 1), 6)
        gather_descr(b).start()

    idx_descr(0, 0).start()
    idx_descr(1, 1).start()
    idx_descr(2, 2).start()
    idx_descr(0, 0).wait()
    prep_and_fire_gather(0)
    idx_descr(1, 1).wait()
    prep_and_fire_gather(1)

    iota = lax.iota(jnp.int32, L)

    def outer(o, carry):
        for b in range(NBUF):
            t = o * NBUF + b

            @pl.when(t + 3 < SEQ_LEN)
            def _():
                idx_descr(t + 3, (b + 3) % NBUF).start()

            @pl.when(t + 2 < SEQ_LEN)
            def _():
                nb = (b + 2) % NBUF
                idx_descr(t + 2, nb).wait()
                prep_and_fire_gather(nb)

            gather_descr(b).wait()

            @pl.when(t >= NBUF)
            def _():
                for d in wb_descrs(t - NBUF, b):
                    d.wait()

            t_vec = jnp.full((L,), t, jnp.int32)

            def e_body(e, carry2):
                ps = plsc.load_gather(pos_v, [t_vec, jnp.full((L,), e, jnp.int32)])
                er = lax.shift_right_logical(e, 3)
                es = jnp.bitwise_and(e, 7)
                for j in range(BBLK // L):
                    sl = pl.ds(j * L, L)
                    rowi = iota + (j * L)
                    coli = hcol_v[b][sl] + e
                    vals = plsc.load_gather(rows_v[b], [rowi, coli])
                    outb_v[b][er, es, sl] = vals * SCALE + ps
                return carry2

            lax.fori_loop(0, EMBED_DIM, e_body, 0, unroll=4)

            for d in wb_descrs(t, b):
                d.start()
        return carry

    lax.fori_loop(0, SEQ_LEN // NBUF, outer, 0)

    for b in range(NBUF):
        for d in wb_descrs(SEQ_LEN - NBUF + b, b):
            d.wait()


def kernel(inputs, token_table, pos_table):
    idx4 = inputs.T.reshape(TROW, 8, TCOL, 128).transpose(0, 2, 1, 3)
    pairs = token_table.reshape(VOCAB // 2, 128)
    out5 = _embed_kernel(idx4, pairs, pos_table)
    return out5.transpose(2, 4, 0, 1, 3).reshape(BATCH, SEQ_LEN, EMBED_DIM)


# speedup vs baseline: 2.0229x; 2.0182x over previous
import functools

import jax
import jax.numpy as jnp
from jax import lax
from jax.experimental import pallas as pl
from jax.experimental.pallas import tpu as pltpu
from jax.experimental.pallas import tpu_sc as plsc

SEQ_LEN = 200
EMBED_DIM = 64
BATCH = 4096
VOCAB = 1000000

NC, NS, L = 2, 16, 16
NW = NC * NS
BBLK = BATCH // NW
TROW, TCOL = SEQ_LEN // 8, BATCH // 128
NBUF = 4
SCALE = 8.0


@functools.partial(
    pl.kernel,
    out_type=jax.ShapeDtypeStruct((SEQ_LEN, 8, TCOL, 8, 128), jnp.float32),
    mesh=plsc.VectorSubcoreMesh(core_axis_name="c", subcore_axis_name="s"),
    compiler_params=pltpu.CompilerParams(
        use_tc_tiling_on_sc=False, needs_layout_passes=False),
    scratch_types=[
        [pltpu.VMEM((BBLK,), jnp.int32) for _ in range(NBUF)],
        [pltpu.VMEM((BBLK,), jnp.int32) for _ in range(NBUF)],
        [pltpu.VMEM((BBLK,), jnp.int32) for _ in range(NBUF)],
        [pltpu.VMEM((BBLK, 128), jnp.float32) for _ in range(NBUF)],
        [pltpu.VMEM((8, 8, 128), jnp.float32) for _ in range(NBUF)],
        pltpu.VMEM((SEQ_LEN, EMBED_DIM), jnp.float32),
        [pltpu.SemaphoreType.DMA for _ in range(NBUF)],
        [pltpu.SemaphoreType.DMA for _ in range(NBUF)],
        [pltpu.SemaphoreType.DMA for _ in range(NBUF)],
    ],
)
def _embed_kernel(idx4_hbm, pairs_hbm, pos_hbm, out_hbm,
                  idx_v, idxp_v, hcol_v, rows_v, outb_v, pos_v,
                  isem, gsem, wsem):
    wid = lax.axis_index("s") * NC + lax.axis_index("c")

    pltpu.sync_copy(pos_hbm, pos_v)

    def idx_descr(t, b):
        return pltpu.make_async_copy(
            idx4_hbm.at[t // 8, wid, t % 8], idx_v[b], isem[b])

    def gather_descr(b):
        return pltpu.make_async_copy(
            pairs_hbm.at[idxp_v[b]], rows_v[b], gsem[b])

    def wb_descrs(t, b):
        return [pltpu.make_async_copy(
            outb_v[b].at[er], out_hbm.at[t, er, wid], wsem[b])
            for er in range(8)]

    def prep_and_fire_gather(b):
        for j in range(BBLK // L):
            sl = pl.ds(j * L, L)
            v = idx_v[b][sl]
            idxp_v[b][sl] = lax.shift_right_logical(v, 1)
            hcol_v[b][sl] = lax.shift_left(jnp.bitwise_and(v, 1), 6)
        gather_descr(b).start()

    idx_descr(0, 0).start()
    idx_descr(1, 1).start()
    idx_descr(2, 2).start()
    idx_descr(0, 0).wait()
    prep_and_fire_gather(0)
    idx_descr(1, 1).wait()
    prep_and_fire_gather(1)

    iota = lax.iota(jnp.int32, L)

    def outer(o, carry):
        for b in range(NBUF):
            t = o * NBUF + b

            @pl.when(t + 3 < SEQ_LEN)
            def _():
                idx_descr(t + 3, (b + 3) % NBUF).start()

            @pl.when(t + 2 < SEQ_LEN)
            def _():
                nb = (b + 2) % NBUF
                idx_descr(t + 2, nb).wait()
                prep_and_fire_gather(nb)

            gather_descr(b).wait()

            @pl.when(t >= NBUF)
            def _():
                for d in wb_descrs(t - NBUF, b):
                    d.wait()

            t_vec = jnp.full((L,), t, jnp.int32)

            @plsc.parallel_loop(0, EMBED_DIM, unroll=4)
            def _(e):
                ps = plsc.load_gather(pos_v, [t_vec, jnp.full((L,), e, jnp.int32)])
                er = lax.shift_right_logical(e, 3)
                es = jnp.bitwise_and(e, 7)
                for j in range(BBLK // L):
                    sl = pl.ds(j * L, L)
                    rowi = iota + (j * L)
                    coli = hcol_v[b][sl] + e
                    vals = plsc.load_gather(rows_v[b], [rowi, coli])
                    outb_v[b][er, es, sl] = vals * SCALE + ps

            for d in wb_descrs(t, b):
                d.start()
        return carry

    lax.fori_loop(0, SEQ_LEN // NBUF, outer, 0)

    for b in range(NBUF):
        for d in wb_descrs(SEQ_LEN - NBUF + b, b):
            d.wait()


def kernel(inputs, token_table, pos_table):
    idx4 = inputs.T.reshape(TROW, 8, TCOL, 128).transpose(0, 2, 1, 3)
    pairs = token_table.reshape(VOCAB // 2, 128)
    out5 = _embed_kernel(idx4, pairs, pos_table)
    return out5.transpose(2, 4, 0, 1, 3).reshape(BATCH, SEQ_LEN, EMBED_DIM)


# speedup vs baseline: 2.1411x; 1.0584x over previous
import functools

import jax
import jax.numpy as jnp
from jax import lax
from jax.experimental import pallas as pl
from jax.experimental.pallas import tpu as pltpu
from jax.experimental.pallas import tpu_sc as plsc

SEQ_LEN = 200
EMBED_DIM = 64
BATCH = 4096
VOCAB = 1000000

NC, NS, L = 2, 16, 16
NW = NC * NS
BBLK = BATCH // NW
TROW, TCOL = SEQ_LEN // 8, BATCH // 128
NBUF = 4
SCALE = 8.0


@functools.partial(
    pl.kernel,
    out_type=jax.ShapeDtypeStruct((SEQ_LEN, 8, TCOL, 8, 128), jnp.float32),
    mesh=plsc.VectorSubcoreMesh(core_axis_name="c", subcore_axis_name="s"),
    compiler_params=pltpu.CompilerParams(
        use_tc_tiling_on_sc=False, needs_layout_passes=False),
    scratch_types=[
        [pltpu.VMEM((BBLK,), jnp.int32) for _ in range(NBUF)],
        [pltpu.VMEM((BBLK, EMBED_DIM), jnp.float32) for _ in range(NBUF)],
        [pltpu.VMEM((8, 8, 128), jnp.float32) for _ in range(NBUF)],
        pltpu.VMEM((SEQ_LEN, EMBED_DIM), jnp.float32),
        [pltpu.SemaphoreType.DMA for _ in range(NBUF)],
        [pltpu.SemaphoreType.DMA for _ in range(NBUF)],
        [pltpu.SemaphoreType.DMA for _ in range(NBUF)],
    ],
)
def _embed_kernel(idx4_hbm, table_hbm, pos_hbm, out_hbm,
                  idx_v, rows_v, outb_v, pos_v,
                  isem, gsem, wsem):
    wid = lax.axis_index("s") * NC + lax.axis_index("c")

    pltpu.sync_copy(pos_hbm, pos_v)

    def idx_descr(t, b):
        return pltpu.make_async_copy(
            idx4_hbm.at[t // 8, wid, t % 8], idx_v[b], isem[b])

    def gather_descr(b):
        return pltpu.make_async_copy(
            table_hbm.at[idx_v[b]], rows_v[b], gsem[b])

    def wb_descrs(t, b):
        return [pltpu.make_async_copy(
            outb_v[b].at[er], out_hbm.at[t, er, wid], wsem[b])
            for er in range(8)]

    idx_descr(0, 0).start()
    idx_descr(1, 1).start()
    idx_descr(2, 2).start()
    idx_descr(0, 0).wait()
    gather_descr(0).start()
    idx_descr(1, 1).wait()
    gather_descr(1).start()

    iota = lax.iota(jnp.int32, L)

    def outer(o, carry):
        for b in range(NBUF):
            t = o * NBUF + b

            @pl.when(t + 3 < SEQ_LEN)
            def _():
                idx_descr(t + 3, (b + 3) % NBUF).start()

            @pl.when(t + 2 < SEQ_LEN)
            def _():
                nb = (b + 2) % NBUF
                idx_descr(t + 2, nb).wait()
                gather_descr(nb).start()

            gather_descr(b).wait()

            @pl.when(t >= NBUF)
            def _():
                for d in wb_descrs(t - NBUF, b):
                    d.wait()

            t_vec = jnp.full((L,), t, jnp.int32)

            @plsc.parallel_loop(0, EMBED_DIM, unroll=4)
            def _(e):
                e_vec = jnp.full((L,), e, jnp.int32)
                ps = plsc.load_gather(pos_v, [t_vec, e_vec])
                er = lax.shift_right_logical(e, 3)
                es = jnp.bitwise_and(e, 7)
                for j in range(BBLK // L):
                    rowi = iota + (j * L)
                    vals = plsc.load_gather(rows_v[b], [rowi, e_vec])
                    outb_v[b][er, es, pl.ds(j * L, L)] = vals * SCALE + ps

            for d in wb_descrs(t, b):
                d.start()
        return carry

    lax.fori_loop(0, SEQ_LEN // NBUF, outer, 0)

    for b in range(NBUF):
        for d in wb_descrs(SEQ_LEN - NBUF + b, b):
            d.wait()


def kernel(inputs, token_table, pos_table):
    idx4 = inputs.T.reshape(TROW, 8, TCOL, 128).transpose(0, 2, 1, 3)
    out5 = _embed_kernel(idx4, token_table, pos_table)
    return out5.transpose(2, 4, 0, 1, 3).reshape(BATCH, SEQ_LEN, EMBED_DIM)
